# baseline (speedup 1.0000x reference)
import jax
import jax.numpy as jnp
from jax.experimental import pallas as pl

N = 4096
D = N + 1
R = 512
G = N // R


def _node_coeffs_all(l, u):
    safe = lambda x: jnp.where(x == 0, jnp.ones_like(x), x)
    lam = u / safe(u - l)
    alpha_c = jnp.where(u < -l, 1e-5, 1.0)
    au_h = jnp.where(u - 6.0 < 6.0 - l, 6.0 / safe(6.0 - l), 1e-5)
    al_h = jnp.where(u < -l, 1e-5, 6.0 / safe(u))
    lam_m = (6.0 - l) / safe(u - l)
    alpha_m = jnp.where(u - 6.0 < 6.0 - l, 1.0, 1e-5)
    m_pos = (u > 0) & (u <= 6) & (l >= 0)
    m_cross = (u > 0) & (u <= 6) & (l < 0)
    m_hcross = (u > 6) & (l <= 0)
    m_mid = (u > 6) & (l > 0) & (l <= 6)
    m_sat = (u > 6) & (l > 6)
    diag_low = jnp.where(m_pos, 1.0, jnp.where(m_cross, alpha_c, jnp.where(m_hcross, al_h, jnp.where(m_mid, lam_m, 0.0))))
    bias_low = jnp.where(m_mid, l * (1.0 - lam_m), jnp.where(m_sat, 6.0, 0.0))
    diag_up = jnp.where(m_pos, 1.0, jnp.where(m_cross, lam, jnp.where(m_hcross, au_h, jnp.where(m_mid, alpha_m, 0.0))))
    bias_up = jnp.where(m_cross, -lam * l, jnp.where(m_hcross, 6.0 * (1.0 - au_h), jnp.where(m_mid, 6.0 * (1.0 - alpha_m), jnp.where(m_sat, 6.0, 0.0))))
    clb = jnp.where(m_pos, l, jnp.where(m_cross, alpha_c * l, jnp.where(m_hcross, al_h * l, jnp.where(m_mid, l, jnp.where(m_sat, 6.0, 0.0)))))
    cub = jnp.where(m_pos, u, jnp.where(m_cross, u, jnp.where(m_hcross, 6.0 + au_h * (u - 6.0), jnp.where(m_mid | m_sat, 6.0, 0.0))))
    return diag_low, bias_low, diag_up, bias_up, clb, cub


def _fused_kernel(l_blk, u_blk, l_full, u_full,
                  alb_ref, aub_ref, clb_ref, cub_ref):
    i = pl.program_id(0)

    @pl.when(i < G)
    def _main():
        dl, _, du, _, _, _ = _node_coeffs_all(l_blk[...], u_blk[...])
        alb_ref[...] = jnp.zeros((R, D), jnp.float32)
        aub_ref[...] = jnp.zeros((R, D), jnp.float32)
        r0 = jax.lax.broadcasted_iota(jnp.int32, (R, R), 0)
        r1 = jax.lax.broadcasted_iota(jnp.int32, (R, R), 1)
        on_diag = r0 == r1
        alb_ref[:, pl.ds(i * R, R)] = jnp.where(on_diag, dl, 0.0)
        aub_ref[:, pl.ds(i * R, R)] = jnp.where(on_diag, du, 0.0)

    @pl.when(i == 0)
    def _concrete():
        _, _, _, _, clb, cub = _node_coeffs_all(l_full[...], u_full[...])
        clb_ref[...] = clb
        cub_ref[...] = cub

    @pl.when(i == G)
    def _bias_row():
        _, bias_low, _, bias_up, _, _ = _node_coeffs_all(l_full[...], u_full[...])
        alb_ref[:, :N] = jnp.broadcast_to(bias_low, (R, N))
        alb_ref[:, N:] = jnp.ones((R, 1), jnp.float32)
        aub_ref[:, :N] = jnp.broadcast_to(bias_up, (R, N))
        aub_ref[:, N:] = jnp.ones((R, 1), jnp.float32)


def kernel(concrete_lower, concrete_upper, abstract_lower_in, abstract_upper_in):
    l_row = concrete_lower.reshape(1, N)
    u_row = concrete_upper.reshape(1, N)

    alb, aub, clb, cub = pl.pallas_call(
        _fused_kernel,
        grid=(G + 1,),
        in_specs=[
            pl.BlockSpec((1, R), lambda i: (0, i)),
            pl.BlockSpec((1, R), lambda i: (0, i)),
            pl.BlockSpec((1, N), lambda i: (0, 0)),
            pl.BlockSpec((1, N), lambda i: (0, 0)),
        ],
        out_specs=(
            pl.BlockSpec((R, D), lambda i: (i, 0)),
            pl.BlockSpec((R, D), lambda i: (i, 0)),
            pl.BlockSpec((1, N), lambda i: (0, 0)),
            pl.BlockSpec((1, N), lambda i: (0, 0)),
        ),
        out_shape=(
            jax.ShapeDtypeStruct((D, D), jnp.float32),
            jax.ShapeDtypeStruct((D, D), jnp.float32),
            jax.ShapeDtypeStruct((1, N), jnp.float32),
            jax.ShapeDtypeStruct((1, N), jnp.float32),
        ),
    )(l_row, u_row, l_row, u_row)

    return ((clb.reshape(N), cub.reshape(N)), (alb, aub))


# speedup vs baseline: 1.0456x; 1.0456x over previous
import jax
import jax.numpy as jnp
from jax.experimental import pallas as pl

N = 4096
D = N + 1
R = 128
G = N // R


def _node_coeffs_all(l, u):
    safe = lambda x: jnp.where(x == 0, jnp.ones_like(x), x)
    lam = u / safe(u - l)
    alpha_c = jnp.where(u < -l, 1e-5, 1.0)
    au_h = jnp.where(u - 6.0 < 6.0 - l, 6.0 / safe(6.0 - l), 1e-5)
    al_h = jnp.where(u < -l, 1e-5, 6.0 / safe(u))
    lam_m = (6.0 - l) / safe(u - l)
    alpha_m = jnp.where(u - 6.0 < 6.0 - l, 1.0, 1e-5)
    m_pos = (u > 0) & (u <= 6) & (l >= 0)
    m_cross = (u > 0) & (u <= 6) & (l < 0)
    m_hcross = (u > 6) & (l <= 0)
    m_mid = (u > 6) & (l > 0) & (l <= 6)
    m_sat = (u > 6) & (l > 6)
    diag_low = jnp.where(m_pos, 1.0, jnp.where(m_cross, alpha_c, jnp.where(m_hcross, al_h, jnp.where(m_mid, lam_m, 0.0))))
    bias_low = jnp.where(m_mid, l * (1.0 - lam_m), jnp.where(m_sat, 6.0, 0.0))
    diag_up = jnp.where(m_pos, 1.0, jnp.where(m_cross, lam, jnp.where(m_hcross, au_h, jnp.where(m_mid, alpha_m, 0.0))))
    bias_up = jnp.where(m_cross, -lam * l, jnp.where(m_hcross, 6.0 * (1.0 - au_h), jnp.where(m_mid, 6.0 * (1.0 - alpha_m), jnp.where(m_sat, 6.0, 0.0))))
    clb = jnp.where(m_pos, l, jnp.where(m_cross, alpha_c * l, jnp.where(m_hcross, al_h * l, jnp.where(m_mid, l, jnp.where(m_sat, 6.0, 0.0)))))
    cub = jnp.where(m_pos, u, jnp.where(m_cross, u, jnp.where(m_hcross, 6.0 + au_h * (u - 6.0), jnp.where(m_mid | m_sat, 6.0, 0.0))))
    return diag_low, bias_low, diag_up, bias_up, clb, cub


def _fused_kernel(l_blk, u_blk, l_full, u_full,
                  alb_ref, aub_ref, clb_ref, cub_ref):
    i = pl.program_id(0)

    @pl.when(i < G)
    def _main():
        dl, _, du, _, _, _ = _node_coeffs_all(l_blk[...], u_blk[...])
        alb_ref[...] = jnp.zeros((R, D), jnp.float32)
        aub_ref[...] = jnp.zeros((R, D), jnp.float32)
        r0 = jax.lax.broadcasted_iota(jnp.int32, (R, R), 0)
        r1 = jax.lax.broadcasted_iota(jnp.int32, (R, R), 1)
        on_diag = r0 == r1
        alb_ref[:, pl.ds(i * R, R)] = jnp.where(on_diag, dl, 0.0)
        aub_ref[:, pl.ds(i * R, R)] = jnp.where(on_diag, du, 0.0)

    @pl.when(i == 0)
    def _concrete():
        _, _, _, _, clb, cub = _node_coeffs_all(l_full[...], u_full[...])
        clb_ref[...] = clb
        cub_ref[...] = cub

    @pl.when(i == G)
    def _bias_row():
        _, bias_low, _, bias_up, _, _ = _node_coeffs_all(l_full[...], u_full[...])
        alb_ref[:, :N] = jnp.broadcast_to(bias_low, (R, N))
        alb_ref[:, N:] = jnp.ones((R, 1), jnp.float32)
        aub_ref[:, :N] = jnp.broadcast_to(bias_up, (R, N))
        aub_ref[:, N:] = jnp.ones((R, 1), jnp.float32)


def kernel(concrete_lower, concrete_upper, abstract_lower_in, abstract_upper_in):
    l_row = concrete_lower.reshape(1, N)
    u_row = concrete_upper.reshape(1, N)

    alb, aub, clb, cub = pl.pallas_call(
        _fused_kernel,
        grid=(G + 1,),
        in_specs=[
            pl.BlockSpec((1, R), lambda i: (0, i)),
            pl.BlockSpec((1, R), lambda i: (0, i)),
            pl.BlockSpec((1, N), lambda i: (0, 0)),
            pl.BlockSpec((1, N), lambda i: (0, 0)),
        ],
        out_specs=(
            pl.BlockSpec((R, D), lambda i: (i, 0)),
            pl.BlockSpec((R, D), lambda i: (i, 0)),
            pl.BlockSpec((1, N), lambda i: (0, 0)),
            pl.BlockSpec((1, N), lambda i: (0, 0)),
        ),
        out_shape=(
            jax.ShapeDtypeStruct((D, D), jnp.float32),
            jax.ShapeDtypeStruct((D, D), jnp.float32),
            jax.ShapeDtypeStruct((1, N), jnp.float32),
            jax.ShapeDtypeStruct((1, N), jnp.float32),
        ),
    )(l_row, u_row, l_row, u_row)

    return ((clb.reshape(N), cub.reshape(N)), (alb, aub))
